# baseline (speedup 1.0000x reference)
import functools

import jax
import jax.numpy as jnp
from jax import lax
from jax.experimental import pallas as pl
from jax.experimental.pallas import tpu as pltpu
from jax.experimental.pallas import tpu_sc as plsc

SEP_ID = 102
SEQ_LEN = 8192
EMBED_DIM = 128
NUM_CORES = 2
NUM_SUBCORES = 16
LANES = 16
NUM_WORKERS = NUM_CORES * NUM_SUBCORES
ROWS_PER_W = SEQ_LEN // NUM_WORKERS
SCAN_CHUNKS = SEQ_LEN // LANES
SCAN_UNROLL = 8
NCOL = EMBED_DIM // LANES
FILL_UNROLL = 4

_mesh = plsc.VectorSubcoreMesh(core_axis_name="c", subcore_axis_name="s")


@functools.partial(
    pl.kernel,
    mesh=_mesh,
    out_type=jax.ShapeDtypeStruct((SEQ_LEN, EMBED_DIM), jnp.float32),
    scratch_types=[
        pltpu.VMEM((SEQ_LEN,), jnp.int32),
        pltpu.VMEM((2, EMBED_DIM), jnp.float32),
        pltpu.VMEM((ROWS_PER_W, EMBED_DIM), jnp.float32),
    ],
)
def _seg_embed(x_hbm, table_hbm, out_hbm, xv, tablev, rowsv):
    cid = lax.axis_index("c")
    sid = lax.axis_index("s")
    wid = sid * NUM_CORES + cid
    out_base = wid * ROWS_PER_W

    pltpu.sync_copy(table_hbm, tablev)
    pltpu.sync_copy(x_hbm, xv)

    lane = lax.iota(jnp.int32, LANES)

    def scan_body(j, carry):
        acc, gidx = carry
        for u in range(SCAN_UNROLL):
            v = xv[pl.ds((j * SCAN_UNROLL + u) * LANES, LANES)]
            acc = jnp.maximum(acc, jnp.where(v == SEP_ID, gidx, -1))
            gidx = gidx + LANES
        return acc, gidx

    acc, _ = lax.fori_loop(0, SCAN_CHUNKS // SCAN_UNROLL, scan_body,
                           (jnp.full((LANES,), -1, jnp.int32), lane))

    last = acc[0]
    for i in range(1, LANES):
        last = jnp.maximum(last, acc[i])
    input_len = jnp.where(last >= 0, last, SEQ_LEN)

    row0 = [tablev[0, pl.ds(c * LANES, LANES)] for c in range(NCOL)]
    row1 = [tablev[1, pl.ds(c * LANES, LANES)] for c in range(NCOL)]
    diff = [row1[c] - row0[c] for c in range(NCOL)]

    n0 = jnp.clip(input_len - out_base, 0, ROWS_PER_W)
    zero = lane * 0

    def fill_body(j, _):
        for u in range(FILL_UNROLL):
            r = j * FILL_UNROLL + u
            m = jnp.where((zero + r) >= n0, 1, 0).astype(jnp.float32)
            for c in range(NCOL):
                rowsv[r, pl.ds(c * LANES, LANES)] = row0[c] + m * diff[c]
        return 0

    lax.fori_loop(0, ROWS_PER_W // FILL_UNROLL, fill_body, 0)

    pltpu.sync_copy(rowsv, out_hbm.at[pl.ds(out_base, ROWS_PER_W)])


def kernel(x, table):
    return _seg_embed(x, table)
